# Initial kernel scaffold; baseline (speedup 1.0000x reference)
import jax
import jax.numpy as jnp
from jax.experimental import pallas as pl


def kernel(x_src, x_dst, edge_label_index):
    raise NotImplementedError("write your pallas kernel here")


# speedup vs baseline: 2.4490x; 2.4490x over previous
import functools

import jax
import jax.numpy as jnp
from jax import lax
from jax.experimental import pallas as pl
from jax.experimental.pallas import tpu as pltpu
from jax.experimental.pallas import tpu_sc as plsc

N_NODES_ = 10000
N_EDGES_ = 320000
D_ = 128

NC = 2
NS = 16
NW = NC * NS

E_PER_W = N_EDGES_ // NW
CHUNK = 80
N_CHUNKS = E_PER_W // CHUNK


_GATHER_DNUMS = lax.GatherDimensionNumbers(
    offset_dims=(), collapsed_slice_dims=(0,), start_index_map=(0,))


def _lane_perm(t, idx):
    return lax.gather(
        t, idx[:, None], _GATHER_DNUMS, slice_sizes=(1,),
        mode=lax.GatherScatterMode.PROMISE_IN_BOUNDS)


def _lane_sum(t):
    lane = lax.iota(jnp.int32, 16)
    for m in (8, 4, 2, 1):
        t = t + _lane_perm(t, lane ^ m)
    return t


def _dot_chunk(rows_a, rows_b, out_v, out_base):
    lane = lax.iota(jnp.int32, 16)

    def group_body(g, _):
        acc = jnp.zeros((16,), jnp.float32)
        for i in range(16):
            e = g * 16 + i
            parts = [
                rows_a[e, pl.ds(16 * k, 16)] * rows_b[e, pl.ds(16 * k, 16)]
                for k in range(8)
            ]
            s4 = [parts[2 * j] + parts[2 * j + 1] for j in range(4)]
            s2 = [s4[0] + s4[1], s4[2] + s4[3]]
            t = _lane_sum(s2[0] + s2[1])
            acc = jnp.where(lane == i, t, acc)
        out_v[pl.ds(out_base + g * 16, 16)] = acc
        return 0

    lax.fori_loop(0, CHUNK // 16, group_body, 0)


def _edge_decoder_kernel(x_src_hbm, x_dst_hbm, idx_src_hbm, idx_dst_hbm,
                         out_hbm, idx_a_v, idx_b_v, rows_a_v, rows_b_v,
                         out_v, sem_a, sem_b):
    wid = lax.axis_index("s") * NC + lax.axis_index("c")
    base = wid * E_PER_W

    def chunk_body(c, _):
        off = base + c * CHUNK
        pltpu.sync_copy(idx_src_hbm.at[pl.ds(off, CHUNK)], idx_a_v)
        pltpu.sync_copy(idx_dst_hbm.at[pl.ds(off, CHUNK)], idx_b_v)
        cp_a = pltpu.async_copy(x_src_hbm.at[idx_a_v], rows_a_v, sem_a)
        cp_b = pltpu.async_copy(x_dst_hbm.at[idx_b_v], rows_b_v, sem_b)
        cp_a.wait()
        cp_b.wait()
        _dot_chunk(rows_a_v, rows_b_v, out_v, c * CHUNK)
        return 0

    lax.fori_loop(0, N_CHUNKS, chunk_body, 0)
    pltpu.sync_copy(out_v, out_hbm.at[pl.ds(base, E_PER_W)])


@jax.jit
def _edge_decoder(x_src, x_dst, idx_src, idx_dst):
    mesh = plsc.VectorSubcoreMesh(core_axis_name="c", subcore_axis_name="s")
    kfn = functools.partial(
        pl.kernel,
        mesh=mesh,
        out_type=jax.ShapeDtypeStruct((N_EDGES_,), jnp.float32),
        scratch_types=[
            pltpu.VMEM((CHUNK,), jnp.int32),
            pltpu.VMEM((CHUNK,), jnp.int32),
            pltpu.VMEM((CHUNK, D_), jnp.float32),
            pltpu.VMEM((CHUNK, D_), jnp.float32),
            pltpu.VMEM((E_PER_W,), jnp.float32),
            pltpu.SemaphoreType.DMA,
            pltpu.SemaphoreType.DMA,
        ],
    )(_edge_decoder_kernel)
    return kfn(x_src, x_dst, idx_src, idx_dst)


def kernel(x_src, x_dst, edge_label_index):
    idx = edge_label_index.astype(jnp.int32)
    return _edge_decoder(x_src, x_dst, idx[0], idx[1])


# speedup vs baseline: 3.8527x; 1.5732x over previous
import functools

import jax
import jax.numpy as jnp
from jax import lax
from jax.experimental import pallas as pl
from jax.experimental.pallas import tpu as pltpu
from jax.experimental.pallas import tpu_sc as plsc

N_NODES_ = 10000
N_EDGES_ = 320000
D_ = 128

NC = 2
NS = 16
NW = NC * NS

E_PER_W = N_EDGES_ // NW
CHUNK = 80
N_CHUNKS = E_PER_W // CHUNK

_GATHER_DNUMS = lax.GatherDimensionNumbers(
    offset_dims=(), collapsed_slice_dims=(0,), start_index_map=(0,))


def _lane_perm(t, idx):
    return lax.gather(
        t, idx[:, None], _GATHER_DNUMS, slice_sizes=(1,),
        mode=lax.GatherScatterMode.PROMISE_IN_BOUNDS)


def _lane_sum(t):
    lane = lax.iota(jnp.int32, 16)
    for m in (8, 4, 2, 1):
        t = t + _lane_perm(t, lane ^ m)
    return t


def _dot_chunk(rows_a, rows_b, out_v, out_base):
    lane = lax.iota(jnp.int32, 16)

    def group_body(g, _):
        acc = jnp.zeros((16,), jnp.float32)
        for i in range(16):
            e = g * 16 + i
            parts = [
                rows_a[e, pl.ds(16 * k, 16)] * rows_b[e, pl.ds(16 * k, 16)]
                for k in range(8)
            ]
            s4 = [parts[2 * j] + parts[2 * j + 1] for j in range(4)]
            s2 = [s4[0] + s4[1], s4[2] + s4[3]]
            t = _lane_sum(s2[0] + s2[1])
            acc = jnp.where(lane == i, t, acc)
        out_v[pl.ds(out_base + g * 16, 16)] = acc
        return 0

    lax.fori_loop(0, CHUNK // 16, group_body, 0)


def _edge_decoder_kernel(x_src_hbm, x_dst_hbm, idx_src_hbm, idx_dst_hbm,
                         out_hbm, ia0, ib0, ia1, ib1,
                         rows_a0, rows_b0, rows_a1, rows_b1,
                         out_v, si0, si1, sa0, sb0, sa1, sb1):
    wid = lax.axis_index("s") * NC + lax.axis_index("c")
    base = wid * E_PER_W

    def issue_idx(c, ia, ib, si):
        off = base + c * CHUNK
        pltpu.async_copy(idx_src_hbm.at[pl.ds(off, CHUNK)], ia, si)
        pltpu.async_copy(idx_dst_hbm.at[pl.ds(off, CHUNK)], ib, si)

    def wait_idx(ia, ib, si):
        pltpu.make_async_copy(idx_src_hbm.at[pl.ds(0, CHUNK)], ia, si).wait()
        pltpu.make_async_copy(idx_dst_hbm.at[pl.ds(0, CHUNK)], ib, si).wait()

    def issue_rows(ia, ib, ra, rb, sa, sb):
        pltpu.async_copy(x_src_hbm.at[ia], ra, sa)
        pltpu.async_copy(x_dst_hbm.at[ib], rb, sb)

    def wait_rows(ia, ib, ra, rb, sa, sb):
        pltpu.make_async_copy(x_src_hbm.at[ia], ra, sa).wait()
        pltpu.make_async_copy(x_dst_hbm.at[ib], rb, sb).wait()

    pltpu.sync_copy(idx_src_hbm.at[pl.ds(base, CHUNK)], ia0)
    pltpu.sync_copy(idx_dst_hbm.at[pl.ds(base, CHUNK)], ib0)
    issue_rows(ia0, ib0, rows_a0, rows_b0, sa0, sb0)
    issue_idx(1, ia1, ib1, si1)

    def pair_body(g, _):
        c0 = 2 * g
        wait_idx(ia1, ib1, si1)
        issue_rows(ia1, ib1, rows_a1, rows_b1, sa1, sb1)
        wait_rows(ia0, ib0, rows_a0, rows_b0, sa0, sb0)
        issue_idx(c0 + 2, ia0, ib0, si0)
        _dot_chunk(rows_a0, rows_b0, out_v, c0 * CHUNK)
        wait_idx(ia0, ib0, si0)
        issue_rows(ia0, ib0, rows_a0, rows_b0, sa0, sb0)
        wait_rows(ia1, ib1, rows_a1, rows_b1, sa1, sb1)

        @pl.when(g < (N_CHUNKS - 3) // 2)
        def _():
            issue_idx(c0 + 3, ia1, ib1, si1)

        _dot_chunk(rows_a1, rows_b1, out_v, (c0 + 1) * CHUNK)
        return 0

    lax.fori_loop(0, (N_CHUNKS - 1) // 2, pair_body, 0)

    wait_rows(ia0, ib0, rows_a0, rows_b0, sa0, sb0)
    _dot_chunk(rows_a0, rows_b0, out_v, (N_CHUNKS - 1) * CHUNK)

    pltpu.sync_copy(out_v, out_hbm.at[pl.ds(base, E_PER_W)])


@jax.jit
def _edge_decoder(x_src, x_dst, idx_src, idx_dst):
    mesh = plsc.VectorSubcoreMesh(core_axis_name="c", subcore_axis_name="s")
    kfn = functools.partial(
        pl.kernel,
        mesh=mesh,
        out_type=jax.ShapeDtypeStruct((N_EDGES_,), jnp.float32),
        scratch_types=[
            pltpu.VMEM((CHUNK,), jnp.int32),
            pltpu.VMEM((CHUNK,), jnp.int32),
            pltpu.VMEM((CHUNK,), jnp.int32),
            pltpu.VMEM((CHUNK,), jnp.int32),
            pltpu.VMEM((CHUNK, D_), jnp.float32),
            pltpu.VMEM((CHUNK, D_), jnp.float32),
            pltpu.VMEM((CHUNK, D_), jnp.float32),
            pltpu.VMEM((CHUNK, D_), jnp.float32),
            pltpu.VMEM((E_PER_W,), jnp.float32),
            pltpu.SemaphoreType.DMA,
            pltpu.SemaphoreType.DMA,
            pltpu.SemaphoreType.DMA,
            pltpu.SemaphoreType.DMA,
            pltpu.SemaphoreType.DMA,
            pltpu.SemaphoreType.DMA,
        ],
    )(_edge_decoder_kernel)
    return kfn(x_src, x_dst, idx_src, idx_dst)


def kernel(x_src, x_dst, edge_label_index):
    idx = edge_label_index.astype(jnp.int32)
    return _edge_decoder(x_src, x_dst, idx[0], idx[1])


# speedup vs baseline: 9.3486x; 2.4265x over previous
import functools

import jax
import jax.numpy as jnp
from jax import lax
from jax.experimental import pallas as pl
from jax.experimental.pallas import tpu as pltpu
from jax.experimental.pallas import tpu_sc as plsc

N_NODES_ = 10000
N_EDGES_ = 320000
D_ = 128

NC = 2
NS = 16
NW = NC * NS

E_PER_W = N_EDGES_ // NW
CHUNK = 80
N_CHUNKS = E_PER_W // CHUNK

_GATHER_DNUMS = lax.GatherDimensionNumbers(
    offset_dims=(), collapsed_slice_dims=(0,), start_index_map=(0,))


def _lane_perm(t, idx):
    return lax.gather(
        t, idx[:, None], _GATHER_DNUMS, slice_sizes=(1,),
        mode=lax.GatherScatterMode.PROMISE_IN_BOUNDS)


def _lane_sum(t):
    lane = lax.iota(jnp.int32, 16)
    for m in (8, 4, 2, 1):
        t = t + _lane_perm(t, lane ^ m)
    return t


def _dot_chunk(rows_a, rows_b, tmp_v, out_v, out_base):
    lane = lax.iota(jnp.int32, 16)

    @plsc.parallel_loop(0, CHUNK, 1, unroll=2)
    def _(e):
        parts = [
            rows_a[e, pl.ds(16 * k, 16)] * rows_b[e, pl.ds(16 * k, 16)]
            for k in range(8)
        ]
        s4 = [parts[2 * j] + parts[2 * j + 1] for j in range(4)]
        s2 = [s4[0] + s4[1], s4[2] + s4[3]]
        t = _lane_sum(s2[0] + s2[1])
        tmp_v[pl.ds(e * 16, 16)] = t

    def compact_body(g, _):
        acc = jnp.zeros((16,), jnp.float32)
        for i in range(16):
            acc = jnp.where(lane == i, tmp_v[pl.ds((g * 16 + i) * 16, 16)],
                            acc)
        out_v[pl.ds(out_base + g * 16, 16)] = acc
        return 0

    lax.fori_loop(0, CHUNK // 16, compact_body, 0)


def _edge_decoder_kernel(x_src_hbm, x_dst_hbm, idx_src_hbm, idx_dst_hbm,
                         out_hbm, ia0, ib0, ia1, ib1,
                         rows_a0, rows_b0, rows_a1, rows_b1,
                         tmp_v, out_v, si0, si1, sa0, sb0, sa1, sb1):
    wid = lax.axis_index("s") * NC + lax.axis_index("c")
    base = wid * E_PER_W

    def issue_idx(c, ia, ib, si):
        off = base + c * CHUNK
        pltpu.async_copy(idx_src_hbm.at[pl.ds(off, CHUNK)], ia, si)
        pltpu.async_copy(idx_dst_hbm.at[pl.ds(off, CHUNK)], ib, si)

    def wait_idx(ia, ib, si):
        pltpu.make_async_copy(idx_src_hbm.at[pl.ds(0, CHUNK)], ia, si).wait()
        pltpu.make_async_copy(idx_dst_hbm.at[pl.ds(0, CHUNK)], ib, si).wait()

    def issue_rows(ia, ib, ra, rb, sa, sb):
        pltpu.async_copy(x_src_hbm.at[ia], ra, sa)
        pltpu.async_copy(x_dst_hbm.at[ib], rb, sb)

    def wait_rows(ia, ib, ra, rb, sa, sb):
        pltpu.make_async_copy(x_src_hbm.at[ia], ra, sa).wait()
        pltpu.make_async_copy(x_dst_hbm.at[ib], rb, sb).wait()

    pltpu.sync_copy(idx_src_hbm.at[pl.ds(base, CHUNK)], ia0)
    pltpu.sync_copy(idx_dst_hbm.at[pl.ds(base, CHUNK)], ib0)
    issue_rows(ia0, ib0, rows_a0, rows_b0, sa0, sb0)
    issue_idx(1, ia1, ib1, si1)

    def pair_body(g, _):
        c0 = 2 * g
        wait_idx(ia1, ib1, si1)
        issue_rows(ia1, ib1, rows_a1, rows_b1, sa1, sb1)
        wait_rows(ia0, ib0, rows_a0, rows_b0, sa0, sb0)
        issue_idx(c0 + 2, ia0, ib0, si0)
        _dot_chunk(rows_a0, rows_b0, tmp_v, out_v, c0 * CHUNK)
        wait_idx(ia0, ib0, si0)
        issue_rows(ia0, ib0, rows_a0, rows_b0, sa0, sb0)
        wait_rows(ia1, ib1, rows_a1, rows_b1, sa1, sb1)

        @pl.when(g < (N_CHUNKS - 3) // 2)
        def _():
            issue_idx(c0 + 3, ia1, ib1, si1)

        _dot_chunk(rows_a1, rows_b1, tmp_v, out_v, (c0 + 1) * CHUNK)
        return 0

    lax.fori_loop(0, (N_CHUNKS - 1) // 2, pair_body, 0)

    wait_rows(ia0, ib0, rows_a0, rows_b0, sa0, sb0)
    _dot_chunk(rows_a0, rows_b0, tmp_v, out_v, (N_CHUNKS - 1) * CHUNK)

    pltpu.sync_copy(out_v, out_hbm.at[pl.ds(base, E_PER_W)])


@jax.jit
def _edge_decoder(x_src, x_dst, idx_src, idx_dst):
    mesh = plsc.VectorSubcoreMesh(core_axis_name="c", subcore_axis_name="s")
    kfn = functools.partial(
        pl.kernel,
        mesh=mesh,
        out_type=jax.ShapeDtypeStruct((N_EDGES_,), jnp.float32),
        scratch_types=[
            pltpu.VMEM((CHUNK,), jnp.int32),
            pltpu.VMEM((CHUNK,), jnp.int32),
            pltpu.VMEM((CHUNK,), jnp.int32),
            pltpu.VMEM((CHUNK,), jnp.int32),
            pltpu.VMEM((CHUNK, D_), jnp.float32),
            pltpu.VMEM((CHUNK, D_), jnp.float32),
            pltpu.VMEM((CHUNK, D_), jnp.float32),
            pltpu.VMEM((CHUNK, D_), jnp.float32),
            pltpu.VMEM((CHUNK * 16,), jnp.float32),
            pltpu.VMEM((E_PER_W,), jnp.float32),
            pltpu.SemaphoreType.DMA,
            pltpu.SemaphoreType.DMA,
            pltpu.SemaphoreType.DMA,
            pltpu.SemaphoreType.DMA,
            pltpu.SemaphoreType.DMA,
            pltpu.SemaphoreType.DMA,
        ],
    )(_edge_decoder_kernel)
    return kfn(x_src, x_dst, idx_src, idx_dst)


def kernel(x_src, x_dst, edge_label_index):
    idx = edge_label_index.astype(jnp.int32)
    return _edge_decoder(x_src, x_dst, idx[0], idx[1])
